# Initial kernel scaffold; baseline (speedup 1.0000x reference)
import jax
import jax.numpy as jnp
from jax.experimental import pallas as pl


def kernel(x, W_enc, b_enc, W_dec, b_dec):
    raise NotImplementedError("write your pallas kernel here")


# speedup vs baseline: 6.6349x; 6.6349x over previous
import jax
import jax.numpy as jnp
from jax import lax
from jax.experimental import pallas as pl
from jax.experimental.pallas import tpu as pltpu

D_MODEL = 768
D_SAE = 12288
BATCH = 128
AUX_COEF = 0.03125
BLK = 1024
NBLK = D_SAE // BLK
def _ukeys(v):
    b = lax.bitcast_convert_type(v, jnp.int32)
    m = b >> 31
    return lax.bitcast_convert_type(b ^ (m | jnp.int32(-2147483648)), jnp.uint32)


def _encode_body(x_ref, w_ref, b_ref, pre_ref):
    pre_ref[...] = (
        jnp.dot(x_ref[...], w_ref[...], preferred_element_type=jnp.float32)
        + b_ref[...]
    )


def _encode(x, W_enc, b_enc2d):
    return pl.pallas_call(
        _encode_body,
        grid=(NBLK,),
        in_specs=[
            pl.BlockSpec((BATCH, D_MODEL), lambda i: (0, 0)),
            pl.BlockSpec((D_MODEL, BLK), lambda i: (0, i)),
            pl.BlockSpec((1, BLK), lambda i: (0, i)),
        ],
        out_specs=pl.BlockSpec((BATCH, BLK), lambda i: (0, i)),
        out_shape=jax.ShapeDtypeStruct((BATCH, D_SAE), jnp.float32),
    )(x, W_enc, b_enc2d)


def _thresh_body(pre_ref, t32_ref, t64_ref):
    u = _ukeys(pre_ref[...])

    def body(_, carry):
        t32, t64, bit = carry
        c32 = t32 | bit
        c64 = t64 | bit
        n32 = jnp.sum((u >= c32).astype(jnp.int32), axis=1, keepdims=True)
        n64 = jnp.sum((u >= c64).astype(jnp.int32), axis=1, keepdims=True)
        t32 = jnp.where(n32 >= 32, c32, t32)
        t64 = jnp.where(n64 >= 64, c64, t64)
        return t32, t64, bit >> 1

    z = jnp.zeros((BATCH, 1), jnp.uint32)
    t32, t64, _ = lax.fori_loop(0, 32, body, (z, z, jnp.uint32(0x80000000)))
    t32_ref[...] = t32
    t64_ref[...] = t64


def _thresholds(pre):
    return pl.pallas_call(
        _thresh_body,
        out_shape=(
            jax.ShapeDtypeStruct((BATCH, 1), jnp.uint32),
            jax.ShapeDtypeStruct((BATCH, 1), jnp.uint32),
        ),
    )(pre)


def _decode_body(pre_ref, t32_ref, t64_ref, wd_ref, x_ref, bdec_ref,
                 lat_ref, xhat_ref, loss_ref, aux_ref, acc1, acc2):
    i = pl.program_id(0)

    @pl.when(i == 0)
    def _():
        acc1[...] = jnp.zeros_like(acc1)
        acc2[...] = jnp.zeros_like(acc2)

    pre = pre_ref[...]
    u = _ukeys(pre)
    relu = jnp.maximum(pre, 0.0)
    m1 = u >= t32_ref[...]
    m2 = (u >= t64_ref[...]) & jnp.logical_not(m1)
    lat = jnp.where(m1, relu, 0.0)
    lat_ref[...] = lat

    wd = wd_ref[...]
    norm2 = jnp.sum(wd * wd, axis=1, keepdims=True)
    inv = 1.0 / jnp.maximum(jnp.sqrt(norm2), 1e-12)
    wdn = wd * inv
    acc1[...] += jnp.dot(lat, wdn, preferred_element_type=jnp.float32)
    aux = jnp.where(m2, relu, 0.0)
    acc2[...] += jnp.dot(aux, wdn, preferred_element_type=jnp.float32)

    @pl.when(i == NBLK - 1)
    def _():
        xh = acc1[...] + bdec_ref[...]
        xhat_ref[...] = xh
        d = xh - x_ref[...]
        loss_ref[0, 0] = jnp.mean(d * d)
        a = acc2[...] + d
        aux_ref[0, 0] = AUX_COEF * jnp.mean(a * a)


def _decode(pre, t32, t64, W_dec, x, bdec2d):
    return pl.pallas_call(
        _decode_body,
        grid=(NBLK,),
        in_specs=[
            pl.BlockSpec((BATCH, BLK), lambda i: (0, i)),
            pl.BlockSpec((BATCH, 1), lambda i: (0, 0)),
            pl.BlockSpec((BATCH, 1), lambda i: (0, 0)),
            pl.BlockSpec((BLK, D_MODEL), lambda i: (i, 0)),
            pl.BlockSpec((BATCH, D_MODEL), lambda i: (0, 0)),
            pl.BlockSpec((1, D_MODEL), lambda i: (0, 0)),
        ],
        out_specs=(
            pl.BlockSpec((BATCH, BLK), lambda i: (0, i)),
            pl.BlockSpec((BATCH, D_MODEL), lambda i: (0, 0)),
            pl.BlockSpec(memory_space=pltpu.SMEM, block_shape=(1, 1),
                         index_map=lambda i: (0, 0)),
            pl.BlockSpec(memory_space=pltpu.SMEM, block_shape=(1, 1),
                         index_map=lambda i: (0, 0)),
        ),
        out_shape=(
            jax.ShapeDtypeStruct((BATCH, D_SAE), jnp.float32),
            jax.ShapeDtypeStruct((BATCH, D_MODEL), jnp.float32),
            jax.ShapeDtypeStruct((1, 1), jnp.float32),
            jax.ShapeDtypeStruct((1, 1), jnp.float32),
        ),
        scratch_shapes=[
            pltpu.VMEM((BATCH, D_MODEL), jnp.float32),
            pltpu.VMEM((BATCH, D_MODEL), jnp.float32),
        ],
    )(pre, t32, t64, W_dec, x, bdec2d)


def kernel(x, W_enc, b_enc, W_dec, b_dec):
    pre = _encode(x, W_enc, b_enc.reshape(1, D_SAE))
    t32, t64 = _thresholds(pre)
    latents, x_hat, loss, aux_loss = _decode(
        pre, t32, t64, W_dec, x, b_dec.reshape(1, D_MODEL))
    return x_hat, latents, loss[0, 0], aux_loss[0, 0]
